# Initial kernel scaffold; baseline (speedup 1.0000x reference)
import jax
import jax.numpy as jnp
from jax.experimental import pallas as pl


def kernel(X, edge_index, W1_0, W1_1, b1, W2_0, W2_1, b2, W3_0, W3_1, b3, Wr, br):
    raise NotImplementedError("write your pallas kernel here")


# speedup vs baseline: 30.8046x; 30.8046x over previous
import jax
import jax.numpy as jnp
from jax import lax
from jax.experimental import pallas as pl
from jax.experimental.pallas import tpu as pltpu
from jax.experimental.pallas import tpu_sc as plsc

_B = 2
_WIN = 5
_N = 50000
_E = 800000
_HID = 32

_NS = 16
_CH = 128
_TCH = 392
_DCH = 200
_PADE = _NS * _TCH * _CH
_PADD = _B * _NS * _DCH * _CH
_DUM = 48
_AGN = _N + _DUM
_RPT = _AGN // _NS
_G = 8
_SG = 4
_DG = 8
_BR = 10000


def _mesh():
  return plsc.VectorSubcoreMesh(core_axis_name="c", subcore_axis_name="s")


_SC_PARAMS = pltpu.CompilerParams(use_tc_tiling_on_sc=False)


def _zero_acc(zbuf, acc, base):
  def _z(i, carry):
    pltpu.sync_copy(zbuf, acc.at[pl.ds(base + i * _CH, _CH)])
    return carry

  lax.fori_loop(0, _RPT // _CH, _z, 0)
  rem = _RPT % _CH
  if rem:
    pltpu.sync_copy(zbuf.at[pl.ds(0, rem)],
                    acc.at[pl.ds(base + _RPT - rem, rem)])


def _sc_degree(cols_deg):
  width = 16

  def body(cols_hbm, out_hbm, ones_v, cidx, zbuf, acc, sem):
    c = lax.axis_index("c")
    s = lax.axis_index("s")

    def _fill(i, carry):
      ones_v[i, :] = jnp.ones((16,), jnp.float32)
      zbuf[i, :] = jnp.zeros((16,), jnp.float32)
      return carry

    lax.fori_loop(0, _CH, _fill, 0)
    base = s * _RPT
    _zero_acc(zbuf, acc, base)
    plsc.subcore_barrier()

    def _group(g, carry):
      pltpu.sync_copy(cols_hbm.at[c, s, pl.ds(g * _DG, _DG)], cidx)
      sd = [pltpu.async_copy(ones_v, acc.at[cidx.at[j]], sem, add=True)
            for j in range(_DG)]
      for d in sd:
        d.wait()
      return carry

    lax.fori_loop(0, _DCH // _DG, _group, 0)
    plsc.subcore_barrier()
    pltpu.sync_copy(acc.at[pl.ds(base, _RPT)],
                    out_hbm.at[c, pl.ds(base, _RPT)])

  f = pl.kernel(
      body,
      out_type=jax.ShapeDtypeStruct((_B, _AGN, width), jnp.float32),
      mesh=_mesh(),
      compiler_params=_SC_PARAMS,
      scratch_types=[
          pltpu.VMEM((_CH, width), jnp.float32),
          pltpu.VMEM((_DG, _CH), jnp.int32),
          pltpu.VMEM((_CH, width), jnp.float32),
          pltpu.VMEM_SHARED((_AGN, width), jnp.float32),
          pltpu.SemaphoreType.DMA,
      ],
  )
  return f(cols_deg)


def _sc_aggregate(m, rows_idx, cols_idx, width):
  wreg = width // 16
  dlen = _SG * _CH

  def body(m_hbm, rows_hbm, cols_hbm, out_hbm,
           ridx, cidx, dbuf, acc, gsem, ssem):
    c = lax.axis_index("c")
    s = lax.axis_index("s")

    def _fill(i, carry):
      for t in range(wreg):
        dbuf[i, pl.ds(t * 16, 16)] = jnp.zeros((16,), jnp.float32)
      return carry

    lax.fori_loop(0, dlen, _fill, 0)
    base = s * _RPT

    def _z(i, carry):
      pltpu.sync_copy(dbuf, acc.at[pl.ds(base + i * dlen, dlen)])
      return carry

    lax.fori_loop(0, _RPT // dlen, _z, 0)
    rem = _RPT % dlen
    if rem:
      pltpu.sync_copy(dbuf.at[pl.ds(0, rem)],
                      acc.at[pl.ds(base + _RPT - rem, rem)])
    plsc.subcore_barrier()

    def _group(g, carry):
      pltpu.sync_copy(rows_hbm.at[c, s, pl.ds(g * _G, _G)], ridx)
      pltpu.sync_copy(cols_hbm.at[s, pl.ds(g * _G, _G)], cidx)
      for half in range(_G // _SG):
        gd = [pltpu.async_copy(m_hbm.at[ridx.at[half * _SG + j]],
                               dbuf.at[pl.ds(j * _CH, _CH)], gsem)
              for j in range(_SG)]
        for d in gd:
          d.wait()
        sd = [pltpu.async_copy(dbuf.at[pl.ds(j * _CH, _CH)],
                               acc.at[cidx.at[half * _SG + j]], ssem, add=True)
              for j in range(_SG)]
        for d in sd:
          d.wait()
      return carry

    lax.fori_loop(0, _TCH // _G, _group, 0)
    plsc.subcore_barrier()
    pltpu.sync_copy(acc.at[pl.ds(base, _RPT)],
                    out_hbm.at[c, pl.ds(base, _RPT)])

  f = pl.kernel(
      body,
      out_type=jax.ShapeDtypeStruct((_B, _AGN, width), jnp.float32),
      mesh=_mesh(),
      compiler_params=_SC_PARAMS,
      scratch_types=[
          pltpu.VMEM((_G, _CH), jnp.int32),
          pltpu.VMEM((_G, _CH), jnp.int32),
          pltpu.VMEM((dlen, width), jnp.float32),
          pltpu.VMEM_SHARED((_AGN, width), jnp.float32),
          pltpu.SemaphoreType.DMA,
          pltpu.SemaphoreType.DMA,
      ],
  )
  return f(m, rows_idx, cols_idx)


def _tc_prepare(degp_t, h0):
  def body(deg_ref, h0_ref, dis_ref, m1_ref):
    d = deg_ref[...]
    dsum = d[:, 0:1] + d[:, 1:2]
    pos = dsum > 0
    dis = jnp.where(pos, lax.rsqrt(jnp.where(pos, dsum, 1.0)), 0.0)
    dis_ref[...] = dis
    m = h0_ref[...] * dis
    m1_ref[...] = jnp.concatenate(
        [m, jnp.zeros((_BR, 16 - _WIN), jnp.float32)], axis=1)

  return pl.pallas_call(
      body,
      grid=(_B * _N // _BR,),
      in_specs=[
          pl.BlockSpec((_BR, 2), lambda i: (i % 5, 0)),
          pl.BlockSpec((_BR, _WIN), lambda i: (i, 0)),
      ],
      out_specs=[
          pl.BlockSpec((_BR, 1), lambda i: (i, 0)),
          pl.BlockSpec((_BR, 16), lambda i: (i, 0)),
      ],
      out_shape=[
          jax.ShapeDtypeStruct((_B * _N, 1), jnp.float32),
          jax.ShapeDtypeStruct((_B * _N, 16), jnp.float32),
      ],
  )(degp_t, h0)


def _tc_layer(hp, s3, disf, w0, w1, b, width, wr=None, brd=None):
  p = hp.shape[1]
  last = wr is not None

  def body(hp_ref, s_ref, dis_ref, w0_ref, w1_ref, b_ref, *rest):
    if last:
      wr_ref, br_ref, h_ref, y_ref = rest
    else:
      h_ref, m_ref = rest
    dis = dis_ref[...]
    agg = s_ref[0, :, :p] * dis
    o = (jnp.dot(hp_ref[...], w0_ref[...], preferred_element_type=jnp.float32)
         + jnp.dot(agg, w1_ref[...], preferred_element_type=jnp.float32)
         + b_ref[...])
    h = jnp.tanh(o)
    h_ref[...] = h
    if last:
      y_ref[...] = (jnp.dot(h, wr_ref[...], preferred_element_type=jnp.float32)
                    + br_ref[...])
    else:
      m_ref[...] = h * dis

  in_specs = [
      pl.BlockSpec((_BR, p), lambda i: (i, 0)),
      pl.BlockSpec((1, _BR, width), lambda i: (i // 5, i % 5, 0)),
      pl.BlockSpec((_BR, 1), lambda i: (i, 0)),
      pl.BlockSpec((p, _HID), lambda i: (0, 0)),
      pl.BlockSpec((p, _HID), lambda i: (0, 0)),
      pl.BlockSpec((1, _HID), lambda i: (0, 0)),
  ]
  out_specs = [pl.BlockSpec((_BR, _HID), lambda i: (i, 0))]
  out_shape = [jax.ShapeDtypeStruct((_B * _N, _HID), jnp.float32)]
  args = [hp, s3, disf, w0, w1, b]
  if last:
    in_specs += [pl.BlockSpec((_HID, 1), lambda i: (0, 0)),
                 pl.BlockSpec((1, 1), lambda i: (0, 0))]
    out_specs += [pl.BlockSpec((_BR, 1), lambda i: (i, 0))]
    out_shape += [jax.ShapeDtypeStruct((_B * _N, 1), jnp.float32)]
    args += [wr, brd]
  else:
    out_specs += [pl.BlockSpec((_BR, _HID), lambda i: (i, 0))]
    out_shape += [jax.ShapeDtypeStruct((_B * _N, _HID), jnp.float32)]

  return pl.pallas_call(
      body,
      grid=(_B * _N // _BR,),
      in_specs=in_specs,
      out_specs=out_specs,
      out_shape=out_shape,
  )(*args)


def kernel(X, edge_index, W1_0, W1_1, b1, W2_0, W2_1, b2, W3_0, W3_1, b3,
           Wr, br):
  h0 = X.reshape(-1, _WIN)
  row = edge_index[0]
  col = edge_index[1]
  pad = _PADE - _E
  rp = jnp.concatenate([row, jnp.zeros((pad,), jnp.int32)])
  cp = jnp.concatenate([col, jnp.full((pad,), _N, jnp.int32)])
  cols_agg = cp.reshape(_NS, _TCH, _CH)
  cols_deg = jnp.concatenate(
      [col, jnp.full((_PADD - _E,), _N, jnp.int32)]).reshape(
          _B, _NS, _DCH, _CH)
  rows_agg = jnp.stack([rp, rp + _N]).reshape(_B, _NS, _TCH, _CH)

  deg = _sc_degree(cols_deg)
  degp_t = jnp.stack([deg[0, :_N, 0], deg[1, :_N, 0]], axis=1)

  disf, m1 = _tc_prepare(degp_t, h0)
  s1 = _sc_aggregate(m1, rows_agg, cols_agg, 16)
  h1, m2 = _tc_layer(h0, s1, disf, W1_0, W1_1, b1.reshape(1, -1), 16)
  s2 = _sc_aggregate(m2, rows_agg, cols_agg, 32)
  h2, m3 = _tc_layer(h1, s2, disf, W2_0, W2_1, b2.reshape(1, -1), 32)
  s3 = _sc_aggregate(m3, rows_agg, cols_agg, 32)
  h3, y = _tc_layer(h2, s3, disf, W3_0, W3_1, b3.reshape(1, -1), 32,
                    Wr, br.reshape(1, 1))
  out = y.reshape(_B, 1, _N)
  return out, h3


# speedup vs baseline: 31.1143x; 1.0101x over previous
import jax
import jax.numpy as jnp
from jax import lax
from jax.experimental import pallas as pl
from jax.experimental.pallas import tpu as pltpu
from jax.experimental.pallas import tpu_sc as plsc

_B = 2
_WIN = 5
_N = 50000
_E = 800000
_HID = 32

_NS = 16
_CH = 128
_TCH = 392
_HT = _TCH // 2
_PADE = _NS * _TCH * _CH
_DUM = 48
_AGN = _N + _DUM
_RPT = _AGN // _NS
_G = 8
_SG = 4
_DG = 14
_BR = 10000


def _mesh():
  return plsc.VectorSubcoreMesh(core_axis_name="c", subcore_axis_name="s")


_SC_PARAMS = pltpu.CompilerParams(use_tc_tiling_on_sc=False)


def _zero_acc(zbuf, acc, base):
  def _z(i, carry):
    pltpu.sync_copy(zbuf, acc.at[pl.ds(base + i * _CH, _CH)])
    return carry

  lax.fori_loop(0, _RPT // _CH, _z, 0)
  rem = _RPT % _CH
  if rem:
    pltpu.sync_copy(zbuf.at[pl.ds(0, rem)],
                    acc.at[pl.ds(base + _RPT - rem, rem)])


def _sc_degree(cols_agg):
  width = 16

  def body(cols_hbm, out_hbm, ones_v, cidx, zbuf, acc, sem):
    c = lax.axis_index("c")
    s = lax.axis_index("s")

    def _fill(i, carry):
      ones_v[i, :] = jnp.ones((16,), jnp.float32)
      zbuf[i, :] = jnp.zeros((16,), jnp.float32)
      return carry

    lax.fori_loop(0, _CH, _fill, 0)
    base = s * _RPT
    _zero_acc(zbuf, acc, base)
    plsc.subcore_barrier()

    def _group(g, carry):
      pltpu.sync_copy(cols_hbm.at[s, pl.ds(c * _HT + g * _DG, _DG)], cidx)
      sd = [pltpu.async_copy(ones_v, acc.at[cidx.at[j]], sem, add=True)
            for j in range(_DG)]
      for d in sd:
        d.wait()
      return carry

    lax.fori_loop(0, _HT // _DG, _group, 0)
    plsc.subcore_barrier()
    pltpu.sync_copy(acc.at[pl.ds(base, _RPT)],
                    out_hbm.at[c, pl.ds(base, _RPT)])

  f = pl.kernel(
      body,
      out_type=jax.ShapeDtypeStruct((_B, _AGN, width), jnp.float32),
      mesh=_mesh(),
      compiler_params=_SC_PARAMS,
      scratch_types=[
          pltpu.VMEM((_CH, width), jnp.float32),
          pltpu.VMEM((_DG, _CH), jnp.int32),
          pltpu.VMEM((_CH, width), jnp.float32),
          pltpu.VMEM_SHARED((_AGN, width), jnp.float32),
          pltpu.SemaphoreType.DMA,
      ],
  )
  return f(cols_agg)


def _sc_aggregate(m, rows_idx, cols_idx, width):
  wreg = width // 16
  dlen = _SG * _CH

  def body(m_hbm, rows_hbm, cols_hbm, out_hbm,
           ridx, cidx, dbuf, acc, gsem, ssem):
    c = lax.axis_index("c")
    s = lax.axis_index("s")

    def _fill(i, carry):
      for t in range(wreg):
        dbuf[i, pl.ds(t * 16, 16)] = jnp.zeros((16,), jnp.float32)
      return carry

    lax.fori_loop(0, dlen, _fill, 0)
    base = s * _RPT

    def _z(i, carry):
      pltpu.sync_copy(dbuf, acc.at[pl.ds(base + i * dlen, dlen)])
      return carry

    lax.fori_loop(0, _RPT // dlen, _z, 0)
    rem = _RPT % dlen
    if rem:
      pltpu.sync_copy(dbuf.at[pl.ds(0, rem)],
                      acc.at[pl.ds(base + _RPT - rem, rem)])
    plsc.subcore_barrier()

    def _group(g, carry):
      pltpu.sync_copy(rows_hbm.at[s, pl.ds(g * _G, _G)], ridx)
      pltpu.sync_copy(cols_hbm.at[s, pl.ds(g * _G, _G)], cidx)
      for half in range(_G // _SG):
        gd = [pltpu.async_copy(m_hbm.at[c].at[ridx.at[half * _SG + j]],
                               dbuf.at[pl.ds(j * _CH, _CH)], gsem)
              for j in range(_SG)]
        for d in gd:
          d.wait()
        sd = [pltpu.async_copy(dbuf.at[pl.ds(j * _CH, _CH)],
                               acc.at[cidx.at[half * _SG + j]], ssem, add=True)
              for j in range(_SG)]
        for d in sd:
          d.wait()
      return carry

    lax.fori_loop(0, _TCH // _G, _group, 0)
    plsc.subcore_barrier()
    pltpu.sync_copy(acc.at[pl.ds(base, _RPT)],
                    out_hbm.at[c, pl.ds(base, _RPT)])

  f = pl.kernel(
      body,
      out_type=jax.ShapeDtypeStruct((_B, _AGN, width), jnp.float32),
      mesh=_mesh(),
      compiler_params=_SC_PARAMS,
      scratch_types=[
          pltpu.VMEM((_G, _CH), jnp.int32),
          pltpu.VMEM((_G, _CH), jnp.int32),
          pltpu.VMEM((dlen, width), jnp.float32),
          pltpu.VMEM_SHARED((_AGN, width), jnp.float32),
          pltpu.SemaphoreType.DMA,
          pltpu.SemaphoreType.DMA,
      ],
  )
  return f(m, rows_idx, cols_idx)


def _tc_prepare(degp_t, h0):
  def body(deg_ref, h0_ref, dis_ref, m1_ref):
    d = deg_ref[...]
    dsum = d[:, 0:1] + d[:, 1:2]
    pos = dsum > 0
    dis = jnp.where(pos, lax.rsqrt(jnp.where(pos, dsum, 1.0)), 0.0)
    dis_ref[...] = dis
    m = h0_ref[...] * dis
    m1_ref[...] = jnp.concatenate(
        [m, jnp.zeros((_BR, 16 - _WIN), jnp.float32)], axis=1)

  return pl.pallas_call(
      body,
      grid=(_B * _N // _BR,),
      in_specs=[
          pl.BlockSpec((_BR, 2), lambda i: (i % 5, 0)),
          pl.BlockSpec((_BR, _WIN), lambda i: (i, 0)),
      ],
      out_specs=[
          pl.BlockSpec((_BR, 1), lambda i: (i, 0)),
          pl.BlockSpec((_BR, 16), lambda i: (i, 0)),
      ],
      out_shape=[
          jax.ShapeDtypeStruct((_B * _N, 1), jnp.float32),
          jax.ShapeDtypeStruct((_B * _N, 16), jnp.float32),
      ],
  )(degp_t, h0)


def _tc_layer(hp, s3, disf, w0, w1, b, width, wr=None, brd=None):
  p = hp.shape[1]
  last = wr is not None

  def body(hp_ref, s_ref, dis_ref, w0_ref, w1_ref, b_ref, *rest):
    if last:
      wr_ref, br_ref, h_ref, y_ref = rest
    else:
      h_ref, m_ref = rest
    dis = dis_ref[...]
    agg = s_ref[0, :, :p] * dis
    o = (jnp.dot(hp_ref[...], w0_ref[...], preferred_element_type=jnp.float32)
         + jnp.dot(agg, w1_ref[...], preferred_element_type=jnp.float32)
         + b_ref[...])
    h = jnp.tanh(o)
    h_ref[...] = h
    if last:
      y_ref[...] = (jnp.dot(h, wr_ref[...], preferred_element_type=jnp.float32)
                    + br_ref[...])
    else:
      m_ref[...] = h * dis

  in_specs = [
      pl.BlockSpec((_BR, p), lambda i: (i, 0)),
      pl.BlockSpec((1, _BR, width), lambda i: (i // 5, i % 5, 0)),
      pl.BlockSpec((_BR, 1), lambda i: (i, 0)),
      pl.BlockSpec((p, _HID), lambda i: (0, 0)),
      pl.BlockSpec((p, _HID), lambda i: (0, 0)),
      pl.BlockSpec((1, _HID), lambda i: (0, 0)),
  ]
  out_specs = [pl.BlockSpec((_BR, _HID), lambda i: (i, 0))]
  out_shape = [jax.ShapeDtypeStruct((_B * _N, _HID), jnp.float32)]
  args = [hp, s3, disf, w0, w1, b]
  if last:
    in_specs += [pl.BlockSpec((_HID, 1), lambda i: (0, 0)),
                 pl.BlockSpec((1, 1), lambda i: (0, 0))]
    out_specs += [pl.BlockSpec((_BR, 1), lambda i: (i, 0))]
    out_shape += [jax.ShapeDtypeStruct((_B * _N, 1), jnp.float32)]
    args += [wr, brd]
  else:
    out_specs += [pl.BlockSpec((_BR, _HID), lambda i: (i, 0))]
    out_shape += [jax.ShapeDtypeStruct((_B * _N, _HID), jnp.float32)]

  return pl.pallas_call(
      body,
      grid=(_B * _N // _BR,),
      in_specs=in_specs,
      out_specs=out_specs,
      out_shape=out_shape,
  )(*args)


def kernel(X, edge_index, W1_0, W1_1, b1, W2_0, W2_1, b2, W3_0, W3_1, b3,
           Wr, br):
  h0 = X.reshape(-1, _WIN)
  row = edge_index[0]
  col = edge_index[1]
  pad = _PADE - _E
  rp = jnp.concatenate([row, jnp.zeros((pad,), jnp.int32)])
  cp = jnp.concatenate([col, jnp.full((pad,), _N, jnp.int32)])
  cols_agg = cp.reshape(_NS, _TCH, _CH)
  rows_agg = rp.reshape(_NS, _TCH, _CH)

  deg = _sc_degree(cols_agg)
  degp_t = jnp.stack([deg[0, :_N, 0], deg[1, :_N, 0]], axis=1)

  disf, m1 = _tc_prepare(degp_t, h0)
  s1 = _sc_aggregate(m1.reshape(_B, _N, 16), rows_agg, cols_agg, 16)
  h1, m2 = _tc_layer(h0, s1, disf, W1_0, W1_1, b1.reshape(1, -1), 16)
  s2 = _sc_aggregate(m2.reshape(_B, _N, _HID), rows_agg, cols_agg, 32)
  h2, m3 = _tc_layer(h1, s2, disf, W2_0, W2_1, b2.reshape(1, -1), 32)
  s3 = _sc_aggregate(m3.reshape(_B, _N, _HID), rows_agg, cols_agg, 32)
  h3, y = _tc_layer(h2, s3, disf, W3_0, W3_1, b3.reshape(1, -1), 32,
                    Wr, br.reshape(1, 1))
  out = y.reshape(_B, 1, _N)
  return out, h3


# speedup vs baseline: 32.1099x; 1.0320x over previous
import jax
import jax.numpy as jnp
from jax import lax
from jax.experimental import pallas as pl
from jax.experimental.pallas import tpu as pltpu
from jax.experimental.pallas import tpu_sc as plsc

_B = 2
_WIN = 5
_N = 50000
_E = 800000
_HID = 32

_NS = 16
_CH = 128
_TCH = 392
_HT = _TCH // 2
_PADE = _NS * _TCH * _CH
_DUM = 48
_AGN = _N + _DUM
_RPT = _AGN // _NS
_G = 8
_SG = 4
_DG = 14
_BR = 10000


def _mesh():
  return plsc.VectorSubcoreMesh(core_axis_name="c", subcore_axis_name="s")


_SC_PARAMS = pltpu.CompilerParams(use_tc_tiling_on_sc=False)


def _zero_acc(zbuf, acc, base):
  def _z(i, carry):
    pltpu.sync_copy(zbuf, acc.at[pl.ds(base + i * _CH, _CH)])
    return carry

  lax.fori_loop(0, _RPT // _CH, _z, 0)
  rem = _RPT % _CH
  if rem:
    pltpu.sync_copy(zbuf.at[pl.ds(0, rem)],
                    acc.at[pl.ds(base + _RPT - rem, rem)])


def _sc_degree(cols_agg):
  width = 16

  def body(cols_hbm, out_hbm, ones_v, cidx, zbuf, acc, sem):
    c = lax.axis_index("c")
    s = lax.axis_index("s")

    def _fill(i, carry):
      ones_v[i, :] = jnp.ones((16,), jnp.float32)
      zbuf[i, :] = jnp.zeros((16,), jnp.float32)
      return carry

    lax.fori_loop(0, _CH, _fill, 0)
    base = s * _RPT
    _zero_acc(zbuf, acc, base)
    plsc.subcore_barrier()

    def _group(g, carry):
      pltpu.sync_copy(cols_hbm.at[s, pl.ds(c * _HT + g * _DG, _DG)], cidx)
      sd = [pltpu.async_copy(ones_v, acc.at[cidx.at[j]], sem, add=True)
            for j in range(_DG)]
      for d in sd:
        d.wait()
      return carry

    lax.fori_loop(0, _HT // _DG, _group, 0)
    plsc.subcore_barrier()
    pltpu.sync_copy(acc.at[pl.ds(base, _RPT)],
                    out_hbm.at[c, pl.ds(base, _RPT)])

  f = pl.kernel(
      body,
      out_type=jax.ShapeDtypeStruct((_B, _AGN, width), jnp.float32),
      mesh=_mesh(),
      compiler_params=_SC_PARAMS,
      scratch_types=[
          pltpu.VMEM((_CH, width), jnp.float32),
          pltpu.VMEM((_DG, _CH), jnp.int32),
          pltpu.VMEM((_CH, width), jnp.float32),
          pltpu.VMEM_SHARED((_AGN, width), jnp.float32),
          pltpu.SemaphoreType.DMA,
      ],
  )
  return f(cols_agg)


def _sc_aggregate(m, rows_idx, cols_idx, width, fused=False):
  wreg = width // 16
  dlen = _SG * _CH
  grp = _SG if fused else _G
  ngrp = (_HT if fused else _TCH) // grp

  def body(m_hbm, rows_hbm, cols_hbm, out_hbm,
           ridx, cidx, dbuf, acc, gsem, ssem):
    c = lax.axis_index("c")
    s = lax.axis_index("s")

    def _fill(i, carry):
      for t in range(wreg):
        dbuf[i, pl.ds(t * 16, 16)] = jnp.zeros((16,), jnp.float32)
      return carry

    lax.fori_loop(0, dlen, _fill, 0)
    base = s * _RPT

    def _z(i, carry):
      pltpu.sync_copy(dbuf, acc.at[pl.ds(base + i * dlen, dlen)])
      return carry

    lax.fori_loop(0, _RPT // dlen, _z, 0)
    rem = _RPT % dlen
    if rem:
      pltpu.sync_copy(dbuf.at[pl.ds(0, rem)],
                      acc.at[pl.ds(base + _RPT - rem, rem)])
    plsc.subcore_barrier()

    def _group(g, carry):
      off = c * _HT + g * grp if fused else g * grp
      pltpu.sync_copy(rows_hbm.at[s, pl.ds(off, grp)], ridx)
      pltpu.sync_copy(cols_hbm.at[s, pl.ds(off, grp)], cidx)
      for half in range(grp // _SG):
        if fused:
          gd = [pltpu.async_copy(m_hbm.at[ridx.at[half * _SG + j]],
                                 dbuf.at[pl.ds(j * _CH, _CH)], gsem)
                for j in range(_SG)]
        else:
          gd = [pltpu.async_copy(m_hbm.at[c].at[ridx.at[half * _SG + j]],
                                 dbuf.at[pl.ds(j * _CH, _CH)], gsem)
                for j in range(_SG)]
        for d in gd:
          d.wait()
        sd = [pltpu.async_copy(dbuf.at[pl.ds(j * _CH, _CH)],
                               acc.at[cidx.at[half * _SG + j]], ssem, add=True)
              for j in range(_SG)]
        for d in sd:
          d.wait()
      return carry

    lax.fori_loop(0, ngrp, _group, 0)
    plsc.subcore_barrier()
    pltpu.sync_copy(acc.at[pl.ds(base, _RPT)],
                    out_hbm.at[c, pl.ds(base, _RPT)])

  f = pl.kernel(
      body,
      out_type=jax.ShapeDtypeStruct((_B, _AGN, width), jnp.float32),
      mesh=_mesh(),
      compiler_params=_SC_PARAMS,
      scratch_types=[
          pltpu.VMEM((grp, _CH), jnp.int32),
          pltpu.VMEM((grp, _CH), jnp.int32),
          pltpu.VMEM((dlen, width), jnp.float32),
          pltpu.VMEM_SHARED((_AGN, width), jnp.float32),
          pltpu.SemaphoreType.DMA,
          pltpu.SemaphoreType.DMA,
      ],
  )
  return f(m, rows_idx, cols_idx)


def _tc_prepare(degp_t, h0):
  def body(deg_ref, h0a_ref, h0b_ref, dis_ref, m1_ref):
    d = deg_ref[...]
    dsum = d[:, 0:1] + d[:, 1:2]
    pos = dsum > 0
    dis = jnp.where(pos, lax.rsqrt(jnp.where(pos, dsum, 1.0)), 0.0)
    dis_ref[...] = dis
    z = jnp.zeros((_BR, 16 - _WIN), jnp.float32)
    m1_ref[...] = jnp.concatenate(
        [h0a_ref[...] * dis, z, h0b_ref[...] * dis, z], axis=1)

  return pl.pallas_call(
      body,
      grid=(_N // _BR,),
      in_specs=[
          pl.BlockSpec((_BR, 2), lambda i: (i, 0)),
          pl.BlockSpec((_BR, _WIN), lambda i: (i, 0)),
          pl.BlockSpec((_BR, _WIN), lambda i: (i + 5, 0)),
      ],
      out_specs=[
          pl.BlockSpec((_BR, 1), lambda i: (i, 0)),
          pl.BlockSpec((_BR, 32), lambda i: (i, 0)),
      ],
      out_shape=[
          jax.ShapeDtypeStruct((_N, 1), jnp.float32),
          jax.ShapeDtypeStruct((_N, 32), jnp.float32),
      ],
  )(degp_t, h0, h0)


def _tc_layer(hp, s3, disf, w0, w1, b, width, wr=None, brd=None, fused=False):
  p = hp.shape[1]
  last = wr is not None
  br = 5000 if fused else _BR
  nb = _N // br

  def body(hp_ref, *refs):
    if fused:
      s0_ref, s1_ref, dis_ref, w0_ref, w1_ref, b_ref, *rest = refs
    else:
      s_ref, dis_ref, w0_ref, w1_ref, b_ref, *rest = refs
    if last:
      wr_ref, br_ref, h_ref, y_ref = rest
    else:
      h_ref, m_ref = rest
    dis = dis_ref[...]
    if fused:
      sb = s0_ref[0] + s1_ref[0]
      agg = jnp.where(pl.program_id(0) >= nb,
                      sb[:, 16:16 + p], sb[:, :p]) * dis
    else:
      agg = s_ref[0, :, :p] * dis
    o = (jnp.dot(hp_ref[...], w0_ref[...], preferred_element_type=jnp.float32)
         + jnp.dot(agg, w1_ref[...], preferred_element_type=jnp.float32)
         + b_ref[...])
    h = jnp.tanh(o)
    h_ref[...] = h
    if last:
      y_ref[...] = (jnp.dot(h, wr_ref[...], preferred_element_type=jnp.float32)
                    + br_ref[...])
    else:
      m_ref[...] = h * dis

  if fused:
    s_specs = [
        pl.BlockSpec((1, br, 32), lambda i: (0, i % nb, 0)),
        pl.BlockSpec((1, br, 32), lambda i: (1, i % nb, 0)),
    ]
    s_args = [s3, s3]
  else:
    s_specs = [pl.BlockSpec((1, br, width), lambda i: (i // nb, i % nb, 0))]
    s_args = [s3]
  in_specs = [
      pl.BlockSpec((br, p), lambda i: (i, 0)),
      *s_specs,
      pl.BlockSpec((br, 1), lambda i: (i % nb, 0)),
      pl.BlockSpec((p, _HID), lambda i: (0, 0)),
      pl.BlockSpec((p, _HID), lambda i: (0, 0)),
      pl.BlockSpec((1, _HID), lambda i: (0, 0)),
  ]
  out_specs = [pl.BlockSpec((br, _HID), lambda i: (i, 0))]
  out_shape = [jax.ShapeDtypeStruct((_B * _N, _HID), jnp.float32)]
  args = [hp, *s_args, disf, w0, w1, b]
  if last:
    in_specs += [pl.BlockSpec((_HID, 1), lambda i: (0, 0)),
                 pl.BlockSpec((1, 1), lambda i: (0, 0))]
    out_specs += [pl.BlockSpec((br, 1), lambda i: (i, 0))]
    out_shape += [jax.ShapeDtypeStruct((_B * _N, 1), jnp.float32)]
    args += [wr, brd]
  else:
    out_specs += [pl.BlockSpec((br, _HID), lambda i: (i, 0))]
    out_shape += [jax.ShapeDtypeStruct((_B * _N, _HID), jnp.float32)]

  return pl.pallas_call(
      body,
      grid=(_B * _N // br,),
      in_specs=in_specs,
      out_specs=out_specs,
      out_shape=out_shape,
  )(*args)


def kernel(X, edge_index, W1_0, W1_1, b1, W2_0, W2_1, b2, W3_0, W3_1, b3,
           Wr, br):
  h0 = X.reshape(-1, _WIN)
  row = edge_index[0]
  col = edge_index[1]
  pad = _PADE - _E
  rp = jnp.concatenate([row, jnp.zeros((pad,), jnp.int32)])
  cp = jnp.concatenate([col, jnp.full((pad,), _N, jnp.int32)])
  cols_agg = cp.reshape(_NS, _TCH, _CH)
  rows_agg = rp.reshape(_NS, _TCH, _CH)

  deg = _sc_degree(cols_agg)
  degp_t = jnp.stack([deg[0, :_N, 0], deg[1, :_N, 0]], axis=1)

  disf, m1 = _tc_prepare(degp_t, h0)
  s1 = _sc_aggregate(m1, rows_agg, cols_agg, 32, fused=True)
  h1, m2 = _tc_layer(h0, s1, disf, W1_0, W1_1, b1.reshape(1, -1), 16,
                     fused=True)
  s2 = _sc_aggregate(m2.reshape(_B, _N, _HID), rows_agg, cols_agg, 32)
  h2, m3 = _tc_layer(h1, s2, disf, W2_0, W2_1, b2.reshape(1, -1), 32)
  s3 = _sc_aggregate(m3.reshape(_B, _N, _HID), rows_agg, cols_agg, 32)
  h3, y = _tc_layer(h2, s3, disf, W3_0, W3_1, b3.reshape(1, -1), 32,
                    Wr, br.reshape(1, 1))
  out = y.reshape(_B, 1, _N)
  return out, h3


# speedup vs baseline: 35.0954x; 1.0930x over previous
import jax
import jax.numpy as jnp
from jax import lax
from jax.experimental import pallas as pl
from jax.experimental.pallas import tpu as pltpu
from jax.experimental.pallas import tpu_sc as plsc

_B = 2
_WIN = 5
_N = 50000
_E = 800000
_HID = 32

_NS = 16
_CH = 128
_TCH = 392
_HT = _TCH // 2
_PADE = _NS * _TCH * _CH
_DUM = 48
_AGN = _N + _DUM
_RPT = _AGN // _NS
_G = 8
_SG = 4
_DG = 14
_BR = 10000


def _mesh():
  return plsc.VectorSubcoreMesh(core_axis_name="c", subcore_axis_name="s")


_SC_PARAMS = pltpu.CompilerParams(use_tc_tiling_on_sc=False)


def _zero_acc(zbuf, acc, base):
  def _z(i, carry):
    pltpu.sync_copy(zbuf, acc.at[pl.ds(base + i * _CH, _CH)])
    return carry

  lax.fori_loop(0, _RPT // _CH, _z, 0)
  rem = _RPT % _CH
  if rem:
    pltpu.sync_copy(zbuf.at[pl.ds(0, rem)],
                    acc.at[pl.ds(base + _RPT - rem, rem)])


def _sc_degree(cols_agg):
  width = 16

  def body(cols_hbm, out_hbm, ones_v, cidx, zbuf, acc, sem):
    c = lax.axis_index("c")
    s = lax.axis_index("s")

    def _fill(i, carry):
      ones_v[i, :] = jnp.ones((16,), jnp.float32)
      zbuf[i, :] = jnp.zeros((16,), jnp.float32)
      return carry

    lax.fori_loop(0, _CH, _fill, 0)
    base = s * _RPT
    _zero_acc(zbuf, acc, base)
    plsc.subcore_barrier()

    def _group(g, carry):
      pltpu.sync_copy(cols_hbm.at[s, pl.ds(c * _HT + g * _DG, _DG)], cidx)
      sd = [pltpu.async_copy(ones_v, acc.at[cidx.at[j]], sem, add=True)
            for j in range(_DG)]
      for d in sd:
        d.wait()
      return carry

    lax.fori_loop(0, _HT // _DG, _group, 0)
    plsc.subcore_barrier()
    pltpu.sync_copy(acc.at[pl.ds(base, _RPT)],
                    out_hbm.at[c, pl.ds(base, _RPT)])

  f = pl.kernel(
      body,
      out_type=jax.ShapeDtypeStruct((_B, _AGN, width), jnp.float32),
      mesh=_mesh(),
      compiler_params=_SC_PARAMS,
      scratch_types=[
          pltpu.VMEM((_CH, width), jnp.float32),
          pltpu.VMEM((_DG, _CH), jnp.int32),
          pltpu.VMEM((_CH, width), jnp.float32),
          pltpu.VMEM_SHARED((_AGN, width), jnp.float32),
          pltpu.SemaphoreType.DMA,
      ],
  )
  return f(cols_agg)


def _sc_aggregate(m, rows_idx, cols_idx, width, fused=False):
  wreg = width // 16
  dlen = _SG * _CH
  grp = 14
  ngrp = (_HT if fused else _TCH) // grp

  def body(m_hbm, rows_hbm, cols_hbm, out_hbm,
           ridx, cidx, dbuf, acc, gsem, ssem):
    c = lax.axis_index("c")
    s = lax.axis_index("s")

    def _fill(i, carry):
      for t in range(wreg):
        dbuf[i, pl.ds(t * 16, 16)] = jnp.zeros((16,), jnp.float32)
      return carry

    lax.fori_loop(0, dlen, _fill, 0)
    base = s * _RPT

    def _z(i, carry):
      pltpu.sync_copy(dbuf, acc.at[pl.ds(base + i * dlen, dlen)])
      return carry

    lax.fori_loop(0, _RPT // dlen, _z, 0)
    rem = _RPT % dlen
    if rem:
      pltpu.sync_copy(dbuf.at[pl.ds(0, rem)],
                      acc.at[pl.ds(base + _RPT - rem, rem)])
    plsc.subcore_barrier()

    def _group(g, carry):
      off = c * _HT + g * grp if fused else g * grp
      pltpu.sync_copy(rows_hbm.at[s, pl.ds(off, grp)], ridx)
      pltpu.sync_copy(cols_hbm.at[s, pl.ds(off, grp)], cidx)
      src = m_hbm if fused else m_hbm.at[c]
      hg = [None] * grp
      hs = [None] * grp
      for j in range(grp):
        if j >= _SG:
          hs[j - _SG].wait()
        hg[j] = pltpu.async_copy(src.at[ridx.at[j]],
                                 dbuf.at[pl.ds((j % _SG) * _CH, _CH)], gsem)
        if j >= 1:
          hg[j - 1].wait()
          hs[j - 1] = pltpu.async_copy(
              dbuf.at[pl.ds(((j - 1) % _SG) * _CH, _CH)],
              acc.at[cidx.at[j - 1]], ssem, add=True)
      hg[grp - 1].wait()
      hs[grp - 1] = pltpu.async_copy(
          dbuf.at[pl.ds(((grp - 1) % _SG) * _CH, _CH)],
          acc.at[cidx.at[grp - 1]], ssem, add=True)
      for j in range(max(0, grp - _SG), grp):
        hs[j].wait()
      return carry

    lax.fori_loop(0, ngrp, _group, 0)
    plsc.subcore_barrier()
    pltpu.sync_copy(acc.at[pl.ds(base, _RPT)],
                    out_hbm.at[c, pl.ds(base, _RPT)])

  f = pl.kernel(
      body,
      out_type=jax.ShapeDtypeStruct((_B, _AGN, width), jnp.float32),
      mesh=_mesh(),
      compiler_params=_SC_PARAMS,
      scratch_types=[
          pltpu.VMEM((grp, _CH), jnp.int32),
          pltpu.VMEM((grp, _CH), jnp.int32),
          pltpu.VMEM((dlen, width), jnp.float32),
          pltpu.VMEM_SHARED((_AGN, width), jnp.float32),
          pltpu.SemaphoreType.DMA,
          pltpu.SemaphoreType.DMA,
      ],
  )
  return f(m, rows_idx, cols_idx)


def _tc_prepare(degp_t, h0):
  def body(deg_ref, h0a_ref, h0b_ref, dis_ref, m1_ref):
    d = deg_ref[...]
    dsum = d[:, 0:1] + d[:, 1:2]
    pos = dsum > 0
    dis = jnp.where(pos, lax.rsqrt(jnp.where(pos, dsum, 1.0)), 0.0)
    dis_ref[...] = dis
    z = jnp.zeros((_BR, 16 - _WIN), jnp.float32)
    m1_ref[...] = jnp.concatenate(
        [h0a_ref[...] * dis, z, h0b_ref[...] * dis, z], axis=1)

  return pl.pallas_call(
      body,
      grid=(_N // _BR,),
      in_specs=[
          pl.BlockSpec((_BR, 2), lambda i: (i, 0)),
          pl.BlockSpec((_BR, _WIN), lambda i: (i, 0)),
          pl.BlockSpec((_BR, _WIN), lambda i: (i + 5, 0)),
      ],
      out_specs=[
          pl.BlockSpec((_BR, 1), lambda i: (i, 0)),
          pl.BlockSpec((_BR, 32), lambda i: (i, 0)),
      ],
      out_shape=[
          jax.ShapeDtypeStruct((_N, 1), jnp.float32),
          jax.ShapeDtypeStruct((_N, 32), jnp.float32),
      ],
  )(degp_t, h0, h0)


def _tc_layer(hp, s3, disf, w0, w1, b, width, wr=None, brd=None, fused=False):
  p = hp.shape[1]
  last = wr is not None
  br = 5000 if fused else _BR
  nb = _N // br

  def body(hp_ref, *refs):
    if fused:
      s0_ref, s1_ref, dis_ref, w0_ref, w1_ref, b_ref, *rest = refs
    else:
      s_ref, dis_ref, w0_ref, w1_ref, b_ref, *rest = refs
    if last:
      wr_ref, br_ref, h_ref, y_ref = rest
    else:
      h_ref, m_ref = rest
    dis = dis_ref[...]
    if fused:
      sb = s0_ref[0] + s1_ref[0]
      agg = jnp.where(pl.program_id(0) >= nb,
                      sb[:, 16:16 + p], sb[:, :p]) * dis
    else:
      agg = s_ref[0, :, :p] * dis
    o = (jnp.dot(hp_ref[...], w0_ref[...], preferred_element_type=jnp.float32)
         + jnp.dot(agg, w1_ref[...], preferred_element_type=jnp.float32)
         + b_ref[...])
    h = jnp.tanh(o)
    h_ref[...] = h
    if last:
      y_ref[...] = (jnp.dot(h, wr_ref[...], preferred_element_type=jnp.float32)
                    + br_ref[...])
    else:
      m_ref[...] = h * dis

  if fused:
    s_specs = [
        pl.BlockSpec((1, br, 32), lambda i: (0, i % nb, 0)),
        pl.BlockSpec((1, br, 32), lambda i: (1, i % nb, 0)),
    ]
    s_args = [s3, s3]
  else:
    s_specs = [pl.BlockSpec((1, br, width), lambda i: (i // nb, i % nb, 0))]
    s_args = [s3]
  in_specs = [
      pl.BlockSpec((br, p), lambda i: (i, 0)),
      *s_specs,
      pl.BlockSpec((br, 1), lambda i: (i % nb, 0)),
      pl.BlockSpec((p, _HID), lambda i: (0, 0)),
      pl.BlockSpec((p, _HID), lambda i: (0, 0)),
      pl.BlockSpec((1, _HID), lambda i: (0, 0)),
  ]
  out_specs = [pl.BlockSpec((br, _HID), lambda i: (i, 0))]
  out_shape = [jax.ShapeDtypeStruct((_B * _N, _HID), jnp.float32)]
  args = [hp, *s_args, disf, w0, w1, b]
  if last:
    in_specs += [pl.BlockSpec((_HID, 1), lambda i: (0, 0)),
                 pl.BlockSpec((1, 1), lambda i: (0, 0))]
    out_specs += [pl.BlockSpec((br, 1), lambda i: (i, 0))]
    out_shape += [jax.ShapeDtypeStruct((_B * _N, 1), jnp.float32)]
    args += [wr, brd]
  else:
    out_specs += [pl.BlockSpec((br, _HID), lambda i: (i, 0))]
    out_shape += [jax.ShapeDtypeStruct((_B * _N, _HID), jnp.float32)]

  return pl.pallas_call(
      body,
      grid=(_B * _N // br,),
      in_specs=in_specs,
      out_specs=out_specs,
      out_shape=out_shape,
  )(*args)


def kernel(X, edge_index, W1_0, W1_1, b1, W2_0, W2_1, b2, W3_0, W3_1, b3,
           Wr, br):
  h0 = X.reshape(-1, _WIN)
  row = edge_index[0]
  col = edge_index[1]
  pad = _PADE - _E
  rp = jnp.concatenate([row, jnp.zeros((pad,), jnp.int32)])
  cp = jnp.concatenate([col, jnp.full((pad,), _N, jnp.int32)])
  cols_agg = cp.reshape(_NS, _TCH, _CH)
  rows_agg = rp.reshape(_NS, _TCH, _CH)

  deg = _sc_degree(cols_agg)
  degp_t = jnp.stack([deg[0, :_N, 0], deg[1, :_N, 0]], axis=1)

  disf, m1 = _tc_prepare(degp_t, h0)
  s1 = _sc_aggregate(m1, rows_agg, cols_agg, 32, fused=True)
  h1, m2 = _tc_layer(h0, s1, disf, W1_0, W1_1, b1.reshape(1, -1), 16,
                     fused=True)
  s2 = _sc_aggregate(m2.reshape(_B, _N, _HID), rows_agg, cols_agg, 32)
  h2, m3 = _tc_layer(h1, s2, disf, W2_0, W2_1, b2.reshape(1, -1), 32)
  s3 = _sc_aggregate(m3.reshape(_B, _N, _HID), rows_agg, cols_agg, 32)
  h3, y = _tc_layer(h2, s3, disf, W3_0, W3_1, b3.reshape(1, -1), 32,
                    Wr, br.reshape(1, 1))
  out = y.reshape(_B, 1, _N)
  return out, h3


# speedup vs baseline: 36.1005x; 1.0286x over previous
import jax
import jax.numpy as jnp
from jax import lax
from jax.experimental import pallas as pl
from jax.experimental.pallas import tpu as pltpu
from jax.experimental.pallas import tpu_sc as plsc

_B = 2
_WIN = 5
_N = 50000
_E = 800000
_HID = 32

_NS = 16
_CH = 128
_TCH = 392
_HT = _TCH // 2
_PADE = _NS * _TCH * _CH
_DUM = 48
_AGN = _N + _DUM
_RPT = _AGN // _NS
_G = 8
_SG = 5
_DG = 14
_BR = 10000


def _mesh():
  return plsc.VectorSubcoreMesh(core_axis_name="c", subcore_axis_name="s")


_SC_PARAMS = pltpu.CompilerParams(use_tc_tiling_on_sc=False)


def _zero_acc(zbuf, acc, base):
  def _z(i, carry):
    pltpu.sync_copy(zbuf, acc.at[pl.ds(base + i * _CH, _CH)])
    return carry

  lax.fori_loop(0, _RPT // _CH, _z, 0)
  rem = _RPT % _CH
  if rem:
    pltpu.sync_copy(zbuf.at[pl.ds(0, rem)],
                    acc.at[pl.ds(base + _RPT - rem, rem)])


def _sc_degree(cols_agg):
  width = 16

  def body(cols_hbm, out_hbm, ones_v, cidx, zbuf, acc, sem):
    c = lax.axis_index("c")
    s = lax.axis_index("s")

    def _fill(i, carry):
      ones_v[i, :] = jnp.ones((16,), jnp.float32)
      zbuf[i, :] = jnp.zeros((16,), jnp.float32)
      return carry

    lax.fori_loop(0, _CH, _fill, 0)
    base = s * _RPT
    _zero_acc(zbuf, acc, base)
    plsc.subcore_barrier()

    def _group(g, carry):
      pltpu.sync_copy(cols_hbm.at[s, pl.ds(c * _HT + g * _DG, _DG)], cidx)
      sd = [pltpu.async_copy(ones_v, acc.at[cidx.at[j]], sem, add=True)
            for j in range(_DG)]
      for d in sd:
        d.wait()
      return carry

    lax.fori_loop(0, _HT // _DG, _group, 0)
    plsc.subcore_barrier()
    pltpu.sync_copy(acc.at[pl.ds(base, _RPT)],
                    out_hbm.at[c, pl.ds(base, _RPT)])

  f = pl.kernel(
      body,
      out_type=jax.ShapeDtypeStruct((_B, _AGN, width), jnp.float32),
      mesh=_mesh(),
      compiler_params=_SC_PARAMS,
      scratch_types=[
          pltpu.VMEM((_CH, width), jnp.float32),
          pltpu.VMEM((_DG, _CH), jnp.int32),
          pltpu.VMEM((_CH, width), jnp.float32),
          pltpu.VMEM_SHARED((_AGN, width), jnp.float32),
          pltpu.SemaphoreType.DMA,
      ],
  )
  return f(cols_agg)


def _sc_aggregate(m, rows_idx, cols_idx, width, fused=False):
  wreg = width // 16
  dlen = _SG * _CH
  grp = 14
  ngrp = (_HT if fused else _TCH) // grp

  def body(m_hbm, rows_hbm, cols_hbm, out_hbm,
           ridx, cidx, dbuf, acc, gsem, ssem):
    c = lax.axis_index("c")
    s = lax.axis_index("s")

    def _fill(i, carry):
      for t in range(wreg):
        dbuf[i, pl.ds(t * 16, 16)] = jnp.zeros((16,), jnp.float32)
      return carry

    lax.fori_loop(0, dlen, _fill, 0)
    base = s * _RPT

    def _z(i, carry):
      pltpu.sync_copy(dbuf, acc.at[pl.ds(base + i * dlen, dlen)])
      return carry

    lax.fori_loop(0, _RPT // dlen, _z, 0)
    rem = _RPT % dlen
    if rem:
      pltpu.sync_copy(dbuf.at[pl.ds(0, rem)],
                      acc.at[pl.ds(base + _RPT - rem, rem)])
    plsc.subcore_barrier()

    def _group(g, carry):
      off = c * _HT + g * grp if fused else g * grp
      pltpu.sync_copy(rows_hbm.at[s, pl.ds(off, grp)], ridx)
      pltpu.sync_copy(cols_hbm.at[s, pl.ds(off, grp)], cidx)
      src = m_hbm if fused else m_hbm.at[c]
      hg = [None] * grp
      hs = [None] * grp
      for j in range(grp):
        if j >= _SG:
          hs[j - _SG].wait()
        hg[j] = pltpu.async_copy(src.at[ridx.at[j]],
                                 dbuf.at[pl.ds((j % _SG) * _CH, _CH)], gsem)
        if j >= 1:
          hg[j - 1].wait()
          hs[j - 1] = pltpu.async_copy(
              dbuf.at[pl.ds(((j - 1) % _SG) * _CH, _CH)],
              acc.at[cidx.at[j - 1]], ssem, add=True)
      hg[grp - 1].wait()
      hs[grp - 1] = pltpu.async_copy(
          dbuf.at[pl.ds(((grp - 1) % _SG) * _CH, _CH)],
          acc.at[cidx.at[grp - 1]], ssem, add=True)
      for j in range(max(0, grp - _SG), grp):
        hs[j].wait()
      return carry

    lax.fori_loop(0, ngrp, _group, 0)
    plsc.subcore_barrier()
    pltpu.sync_copy(acc.at[pl.ds(base, _RPT)],
                    out_hbm.at[c, pl.ds(base, _RPT)])

  f = pl.kernel(
      body,
      out_type=jax.ShapeDtypeStruct((_B, _AGN, width), jnp.float32),
      mesh=_mesh(),
      compiler_params=_SC_PARAMS,
      scratch_types=[
          pltpu.VMEM((grp, _CH), jnp.int32),
          pltpu.VMEM((grp, _CH), jnp.int32),
          pltpu.VMEM((dlen, width), jnp.float32),
          pltpu.VMEM_SHARED((_AGN, width), jnp.float32),
          pltpu.SemaphoreType.DMA,
          pltpu.SemaphoreType.DMA,
      ],
  )
  return f(m, rows_idx, cols_idx)


def _tc_prepare(deg, h0):
  br = 5000
  nb = _N // br

  def body(d0_ref, d1_ref, h0a_ref, h0b_ref, dis_ref, m1_ref):
    dsum = d0_ref[0, :, 0:1] + d1_ref[0, :, 0:1]
    pos = dsum > 0
    dis = jnp.where(pos, lax.rsqrt(jnp.where(pos, dsum, 1.0)), 0.0)
    dis_ref[...] = dis
    z = jnp.zeros((br, 16 - _WIN), jnp.float32)
    m1_ref[...] = jnp.concatenate(
        [h0a_ref[...] * dis, z, h0b_ref[...] * dis, z], axis=1)

  return pl.pallas_call(
      body,
      grid=(nb,),
      in_specs=[
          pl.BlockSpec((1, br, 16), lambda i: (0, i, 0)),
          pl.BlockSpec((1, br, 16), lambda i: (1, i, 0)),
          pl.BlockSpec((br, _WIN), lambda i: (i, 0)),
          pl.BlockSpec((br, _WIN), lambda i: (i + nb, 0)),
      ],
      out_specs=[
          pl.BlockSpec((br, 1), lambda i: (i, 0)),
          pl.BlockSpec((br, 32), lambda i: (i, 0)),
      ],
      out_shape=[
          jax.ShapeDtypeStruct((_N, 1), jnp.float32),
          jax.ShapeDtypeStruct((_N, 32), jnp.float32),
      ],
  )(deg, deg, h0, h0)


def _tc_layer(hp, s3, disf, w0, w1, b, width, wr=None, brd=None, fused=False):
  p = hp.shape[1]
  last = wr is not None
  br = 5000 if fused else _BR
  nb = _N // br

  def body(hp_ref, *refs):
    if fused:
      s0_ref, s1_ref, dis_ref, w0_ref, w1_ref, b_ref, *rest = refs
    else:
      s_ref, dis_ref, w0_ref, w1_ref, b_ref, *rest = refs
    if last:
      wr_ref, br_ref, h_ref, y_ref = rest
    else:
      h_ref, m_ref = rest
    dis = dis_ref[...]
    if fused:
      sb = s0_ref[0] + s1_ref[0]
      agg = jnp.where(pl.program_id(0) >= nb,
                      sb[:, 16:16 + p], sb[:, :p]) * dis
    else:
      agg = s_ref[0, :, :p] * dis
    o = (jnp.dot(hp_ref[...], w0_ref[...], preferred_element_type=jnp.float32)
         + jnp.dot(agg, w1_ref[...], preferred_element_type=jnp.float32)
         + b_ref[...])
    h = jnp.tanh(o)
    h_ref[...] = h
    if last:
      y_ref[...] = (jnp.dot(h, wr_ref[...], preferred_element_type=jnp.float32)
                    + br_ref[...])
    else:
      m_ref[...] = h * dis

  if fused:
    s_specs = [
        pl.BlockSpec((1, br, 32), lambda i: (0, i % nb, 0)),
        pl.BlockSpec((1, br, 32), lambda i: (1, i % nb, 0)),
    ]
    s_args = [s3, s3]
  else:
    s_specs = [pl.BlockSpec((1, br, width), lambda i: (i // nb, i % nb, 0))]
    s_args = [s3]
  in_specs = [
      pl.BlockSpec((br, p), lambda i: (i, 0)),
      *s_specs,
      pl.BlockSpec((br, 1), lambda i: (i % nb, 0)),
      pl.BlockSpec((p, _HID), lambda i: (0, 0)),
      pl.BlockSpec((p, _HID), lambda i: (0, 0)),
      pl.BlockSpec((1, _HID), lambda i: (0, 0)),
  ]
  out_specs = [pl.BlockSpec((br, _HID), lambda i: (i, 0))]
  out_shape = [jax.ShapeDtypeStruct((_B * _N, _HID), jnp.float32)]
  args = [hp, *s_args, disf, w0, w1, b]
  if last:
    in_specs += [pl.BlockSpec((_HID, 1), lambda i: (0, 0)),
                 pl.BlockSpec((1, 1), lambda i: (0, 0))]
    out_specs += [pl.BlockSpec((br, 1), lambda i: (i, 0))]
    out_shape += [jax.ShapeDtypeStruct((_B * _N, 1), jnp.float32)]
    args += [wr, brd]
  else:
    out_specs += [pl.BlockSpec((br, _HID), lambda i: (i, 0))]
    out_shape += [jax.ShapeDtypeStruct((_B * _N, _HID), jnp.float32)]

  return pl.pallas_call(
      body,
      grid=(_B * _N // br,),
      in_specs=in_specs,
      out_specs=out_specs,
      out_shape=out_shape,
  )(*args)


def kernel(X, edge_index, W1_0, W1_1, b1, W2_0, W2_1, b2, W3_0, W3_1, b3,
           Wr, br):
  h0 = X.reshape(-1, _WIN)
  row = edge_index[0]
  col = edge_index[1]
  pad = _PADE - _E
  rp = jnp.concatenate([row, jnp.zeros((pad,), jnp.int32)])
  cp = jnp.concatenate([col, jnp.full((pad,), _N, jnp.int32)])
  cols_agg = cp.reshape(_NS, _TCH, _CH)
  rows_agg = rp.reshape(_NS, _TCH, _CH)

  deg = _sc_degree(cols_agg)
  disf, m1 = _tc_prepare(deg, h0)
  s1 = _sc_aggregate(m1, rows_agg, cols_agg, 32, fused=True)
  h1, m2 = _tc_layer(h0, s1, disf, W1_0, W1_1, b1.reshape(1, -1), 16,
                     fused=True)
  s2 = _sc_aggregate(m2.reshape(_B, _N, _HID), rows_agg, cols_agg, 32)
  h2, m3 = _tc_layer(h1, s2, disf, W2_0, W2_1, b2.reshape(1, -1), 32)
  s3 = _sc_aggregate(m3.reshape(_B, _N, _HID), rows_agg, cols_agg, 32)
  h3, y = _tc_layer(h2, s3, disf, W3_0, W3_1, b3.reshape(1, -1), 32,
                    Wr, br.reshape(1, 1))
  out = y.reshape(_B, 1, _N)
  return out, h3
